# baseline (speedup 1.0000x reference)
import functools

import jax
import jax.numpy as jnp
from jax.experimental import pallas as pl

_K = 100
_NEG = -70.0
_V = 4096
_S_BLK = 256


def _topk_mask_body(x_ref, o_ref):
    x = x_ref[0]
    i = jax.lax.bitcast_convert_type(x, jnp.int32)
    key = jnp.where(i < 0, i ^ jnp.int32(0x7FFFFFFF), i)

    ones_row = jnp.ones((1, _V), dtype=jnp.float32)

    def count_ge(c):
        m = (key >= c).astype(jnp.float32)
        cnt = jnp.dot(ones_row, m, preferred_element_type=jnp.float32)
        return cnt.astype(jnp.int32)

    prefix = jnp.full((1, x.shape[1]), -(2 ** 31), dtype=jnp.int32)
    cand = jnp.zeros_like(prefix)
    prefix = jnp.where(count_ge(cand) >= _K, cand, prefix)
    for b in range(30, -1, -1):
        cand = prefix | jnp.int32(1 << b)
        prefix = jnp.where(count_ge(cand) >= _K, cand, prefix)
    t = prefix

    gt = key > t
    eq = key == t
    gtf = gt.astype(jnp.float32)
    cnt_gt = jnp.dot(ones_row, gtf,
                     preferred_element_type=jnp.float32).astype(jnp.int32)
    n_eq_keep = (_K - cnt_gt).astype(jnp.float32)

    C = 128
    r_i = jax.lax.broadcasted_iota(jnp.int32, (C, C), 0)
    c_i = jax.lax.broadcasted_iota(jnp.int32, (C, C), 1)
    tril = (c_i < r_i).astype(jnp.float32)
    run = jnp.zeros((1, x.shape[1]), dtype=jnp.float32)
    for c in range(_V // C):
        lo, hi = c * C, (c + 1) * C
        eqf_c = eq[lo:hi].astype(jnp.float32)
        pre_c = jnp.dot(tril, eqf_c, preferred_element_type=jnp.float32) + run
        keep_c = gt[lo:hi] | (eq[lo:hi] & (pre_c < n_eq_keep))
        o_ref[0, lo:hi, :] = jnp.where(keep_c, x[lo:hi], _NEG)
        run = run + jnp.sum(eqf_c, axis=0, keepdims=True)


@jax.jit
def _topk_mask(logits):
    B, V, S = logits.shape
    grid = (B, S // _S_BLK)
    return pl.pallas_call(
        _topk_mask_body,
        grid=grid,
        in_specs=[pl.BlockSpec((1, V, _S_BLK), lambda b, s: (b, 0, s))],
        out_specs=pl.BlockSpec((1, V, _S_BLK), lambda b, s: (b, 0, s)),
        out_shape=jax.ShapeDtypeStruct((B, V, S), jnp.float32),
    )(logits)


def kernel(logits, k):
    del k
    return _topk_mask(logits)


# speedup vs baseline: 1.2661x; 1.2661x over previous
import functools

import jax
import jax.numpy as jnp
from jax import lax
from jax.experimental import pallas as pl
from jax.experimental.pallas import tpu as pltpu
from jax.experimental.pallas import tpu_sc as plsc

_K = 100
_NEG = -70.0
_B, _V, _S = 16, 4096, 1024
_B_SC = 8


_S_BLK = 256


def _tc_body(x_ref, o_ref):
    x = x_ref[0]
    i = jax.lax.bitcast_convert_type(x, jnp.int32)
    key = jnp.where(i < 0, i ^ jnp.int32(0x7FFFFFFF), i)

    ones_row = jnp.ones((1, _V), dtype=jnp.float32)

    def count_ge(c):
        m = (key >= c).astype(jnp.float32)
        cnt = jnp.dot(ones_row, m, preferred_element_type=jnp.float32)
        return cnt.astype(jnp.int32)

    prefix = jnp.full((1, x.shape[1]), -(2 ** 31), dtype=jnp.int32)
    cand = jnp.zeros_like(prefix)
    prefix = jnp.where(count_ge(cand) >= _K, cand, prefix)
    for b in range(30, -1, -1):
        cand = prefix | jnp.int32(1 << b)
        prefix = jnp.where(count_ge(cand) >= _K, cand, prefix)
    t = prefix

    gt = key > t
    eq = key == t
    gtf = gt.astype(jnp.float32)
    cnt_gt = jnp.dot(ones_row, gtf,
                     preferred_element_type=jnp.float32).astype(jnp.int32)
    n_eq_keep = (_K - cnt_gt).astype(jnp.float32)

    C = 128
    r_i = jax.lax.broadcasted_iota(jnp.int32, (C, C), 0)
    c_i = jax.lax.broadcasted_iota(jnp.int32, (C, C), 1)
    tril = (c_i < r_i).astype(jnp.float32)
    run = jnp.zeros((1, x.shape[1]), dtype=jnp.float32)
    for c in range(_V // C):
        lo, hi = c * C, (c + 1) * C
        eqf_c = eq[lo:hi].astype(jnp.float32)
        pre_c = jnp.dot(tril, eqf_c, preferred_element_type=jnp.float32) + run
        keep_c = gt[lo:hi] | (eq[lo:hi] & (pre_c < n_eq_keep))
        o_ref[0, lo:hi, :] = jnp.where(keep_c, x[lo:hi], _NEG)
        run = run + jnp.sum(eqf_c, axis=0, keepdims=True)


def _topk_mask_tc(logits):
    grid = (_B - _B_SC, _S // _S_BLK)
    return pl.pallas_call(
        _tc_body,
        grid=grid,
        in_specs=[pl.BlockSpec((1, _V, _S_BLK),
                               lambda b, s: (b + _B_SC, 0, s))],
        out_specs=pl.BlockSpec((1, _V, _S_BLK),
                               lambda b, s: (b + _B_SC, 0, s)),
        out_shape=jax.ShapeDtypeStruct((_B, _V, _S), jnp.float32),
    )(logits)


_LN = 16
_NBINS = 256
_NW = 32
_SC_JOBS = _B_SC * (_S // _LN)
_SC_JPW = _SC_JOBS // _NW


def _key_of(x):
    i = plsc.bitcast(x, jnp.int32)
    m = lax.shift_right_arithmetic(i, 31)
    ui = i ^ (m | jnp.int32(-2147483648))
    return plsc.bitcast(ui, jnp.uint32)


def _sc_body(logits_hbm, out_hbm, x_v, hist_v):
    cid = lax.axis_index("c")
    sid = lax.axis_index("s")
    wid = sid * 2 + cid
    lanes = lax.iota(jnp.int32, _LN)
    ones_i = jnp.ones((_LN,), jnp.int32)
    zero_v = jnp.zeros((_LN,), jnp.int32)

    def do_job(j, carry):
        job = j * _NW + wid
        b = job // (_S // _LN)
        s0 = (job % (_S // _LN)) * _LN
        pltpu.sync_copy(logits_hbm.at[b, :, pl.ds(s0, _LN)], x_v)

        prefix = jnp.zeros((_LN,), jnp.uint32)
        rank = jnp.full((_LN,), _K, jnp.int32)

        for p, shift in enumerate((24, 16, 8, 0)):
            @plsc.parallel_loop(0, _NBINS, unroll=8)
            def _(i):
                hist_v[i] = jnp.zeros((_LN,), jnp.int32)

            sh = jnp.uint32(shift)
            hi_sh = jnp.uint32(shift + 8)
            pref_hi = prefix >> hi_sh

            @plsc.parallel_loop(0, _V, unroll=8)
            def _(v):
                uk = _key_of(x_v[v])
                binv = ((uk >> sh) & jnp.uint32(0xFF)).astype(jnp.int32)
                if p == 0:
                    plsc.addupdate_scatter(hist_v, [binv, lanes], ones_i)
                else:
                    act = (uk >> hi_sh) == pref_hi
                    plsc.addupdate_scatter(hist_v, [binv, lanes], ones_i,
                                           mask=act)

            @plsc.parallel_loop(0, _NBINS, unroll=8,
                                carry=(zero_v, zero_v, zero_v))
            def scan_res(i, c):
                cum, digit, above = c
                r_bin = _NBINS - 1 - i
                h = hist_v[r_bin]
                cum2 = cum + h
                crossed = (cum < rank) & (cum2 >= rank)
                digit = jnp.where(crossed, r_bin, digit)
                above = jnp.where(crossed, cum, above)
                return (cum2, digit, above)

            _, digit, above = scan_res
            prefix = prefix | (digit.astype(jnp.uint32) << sh)
            rank = rank - above

        t_u, n_keep = prefix, rank

        @plsc.parallel_loop(0, _V, unroll=8, carry=zero_v)
        def _(v, cnteq):
            xv = x_v[v]
            uk = _key_of(xv)
            gt = uk > t_u
            eq = uk == t_u
            keep = gt | (eq & (cnteq < n_keep))
            x_v[v] = jnp.where(keep, xv, jnp.float32(_NEG))
            return cnteq + jnp.where(eq, 1, 0)

        pltpu.sync_copy(x_v, out_hbm.at[b, :, pl.ds(s0, _LN)])
        return carry

    lax.fori_loop(0, _SC_JPW, do_job, 0)


def _topk_mask_sc(logits):
    mesh = plsc.VectorSubcoreMesh(core_axis_name="c", subcore_axis_name="s")
    fn = functools.partial(
        pl.kernel,
        mesh=mesh,
        out_type=jax.ShapeDtypeStruct((_B_SC, _V, _S), jnp.float32),
        scratch_types=[pltpu.VMEM((_V, _LN), jnp.float32),
                       pltpu.VMEM((_NBINS, _LN), jnp.int32)],
        compiler_params=pltpu.CompilerParams(use_tc_tiling_on_sc=False,
                                             needs_layout_passes=False),
    )(_sc_body)
    return fn(logits)


@jax.jit
def _topk_mask(logits):
    sc_out = _topk_mask_sc(logits)
    tc_out = _topk_mask_tc(logits)
    return lax.dynamic_update_slice(tc_out, sc_out, (0, 0, 0))


def kernel(logits, k):
    del k
    return _topk_mask(logits)


# speedup vs baseline: 1.2706x; 1.0036x over previous
import functools

import jax
import jax.numpy as jnp
from jax import lax
from jax.experimental import pallas as pl
from jax.experimental.pallas import tpu as pltpu
from jax.experimental.pallas import tpu_sc as plsc

_K = 100
_NEG = -70.0
_B, _V, _S = 16, 4096, 1024
_B_SC = 8


_S_BLK = 512


def _tc_body(x_ref, o_ref):
    x = x_ref[0]
    i = jax.lax.bitcast_convert_type(x, jnp.int32)
    key = jnp.where(i < 0, i ^ jnp.int32(0x7FFFFFFF), i)

    ones_row = jnp.ones((1, _V), dtype=jnp.float32)

    def count_ge(c):
        m = (key >= c).astype(jnp.float32)
        cnt = jnp.dot(ones_row, m, preferred_element_type=jnp.float32)
        return cnt.astype(jnp.int32)

    prefix = jnp.full((1, x.shape[1]), -(2 ** 31), dtype=jnp.int32)
    cand = jnp.zeros_like(prefix)
    prefix = jnp.where(count_ge(cand) >= _K, cand, prefix)
    for b in range(30, -1, -1):
        cand = prefix | jnp.int32(1 << b)
        prefix = jnp.where(count_ge(cand) >= _K, cand, prefix)
    t = prefix

    gt = key > t
    eq = key == t
    gtf = gt.astype(jnp.float32)
    cnt_gt = jnp.dot(ones_row, gtf,
                     preferred_element_type=jnp.float32).astype(jnp.int32)
    n_eq_keep = (_K - cnt_gt).astype(jnp.float32)

    C = 128
    r_i = jax.lax.broadcasted_iota(jnp.int32, (C, C), 0)
    c_i = jax.lax.broadcasted_iota(jnp.int32, (C, C), 1)
    tril = (c_i < r_i).astype(jnp.float32)
    run = jnp.zeros((1, x.shape[1]), dtype=jnp.float32)
    for c in range(_V // C):
        lo, hi = c * C, (c + 1) * C
        eqf_c = eq[lo:hi].astype(jnp.float32)
        pre_c = jnp.dot(tril, eqf_c, preferred_element_type=jnp.float32) + run
        keep_c = gt[lo:hi] | (eq[lo:hi] & (pre_c < n_eq_keep))
        o_ref[0, lo:hi, :] = jnp.where(keep_c, x[lo:hi], _NEG)
        run = run + jnp.sum(eqf_c, axis=0, keepdims=True)


def _topk_mask_tc(logits):
    grid = (_B - _B_SC, _S // _S_BLK)
    return pl.pallas_call(
        _tc_body,
        grid=grid,
        in_specs=[pl.BlockSpec((1, _V, _S_BLK),
                               lambda b, s: (b + _B_SC, 0, s))],
        out_specs=pl.BlockSpec((1, _V, _S_BLK),
                               lambda b, s: (b + _B_SC, 0, s)),
        out_shape=jax.ShapeDtypeStruct((_B, _V, _S), jnp.float32),
    )(logits)


_LN = 16
_NBINS = 256
_NW = 32
_SC_JOBS = _B_SC * (_S // _LN)
_SC_JPW = _SC_JOBS // _NW


def _key_of(x):
    i = plsc.bitcast(x, jnp.int32)
    m = lax.shift_right_arithmetic(i, 31)
    ui = i ^ (m | jnp.int32(-2147483648))
    return plsc.bitcast(ui, jnp.uint32)


def _sc_body(logits_hbm, out_hbm, x_v, hist_v):
    cid = lax.axis_index("c")
    sid = lax.axis_index("s")
    wid = sid * 2 + cid
    lanes = lax.iota(jnp.int32, _LN)
    ones_i = jnp.ones((_LN,), jnp.int32)
    zero_v = jnp.zeros((_LN,), jnp.int32)

    def do_job(j, carry):
        job = j * _NW + wid
        b = job // (_S // _LN)
        s0 = (job % (_S // _LN)) * _LN
        pltpu.sync_copy(logits_hbm.at[b, :, pl.ds(s0, _LN)], x_v)

        prefix = jnp.zeros((_LN,), jnp.uint32)
        rank = jnp.full((_LN,), _K, jnp.int32)

        for p, shift in enumerate((24, 16, 8, 0)):
            @plsc.parallel_loop(0, _NBINS, unroll=8)
            def _(i):
                hist_v[i] = jnp.zeros((_LN,), jnp.int32)

            sh = jnp.uint32(shift)
            hi_sh = jnp.uint32(shift + 8)
            pref_hi = prefix >> hi_sh

            @plsc.parallel_loop(0, _V, unroll=8)
            def _(v):
                uk = _key_of(x_v[v])
                binv = ((uk >> sh) & jnp.uint32(0xFF)).astype(jnp.int32)
                if p == 0:
                    plsc.addupdate_scatter(hist_v, [binv, lanes], ones_i)
                else:
                    act = (uk >> hi_sh) == pref_hi
                    plsc.addupdate_scatter(hist_v, [binv, lanes], ones_i,
                                           mask=act)

            @plsc.parallel_loop(0, _NBINS, unroll=8,
                                carry=(zero_v, zero_v, zero_v))
            def scan_res(i, c):
                cum, digit, above = c
                r_bin = _NBINS - 1 - i
                h = hist_v[r_bin]
                cum2 = cum + h
                crossed = (cum < rank) & (cum2 >= rank)
                digit = jnp.where(crossed, r_bin, digit)
                above = jnp.where(crossed, cum, above)
                return (cum2, digit, above)

            _, digit, above = scan_res
            prefix = prefix | (digit.astype(jnp.uint32) << sh)
            rank = rank - above

        t_u, n_keep = prefix, rank

        @plsc.parallel_loop(0, _V, unroll=8, carry=zero_v)
        def _(v, cnteq):
            xv = x_v[v]
            uk = _key_of(xv)
            gt = uk > t_u
            eq = uk == t_u
            keep = gt | (eq & (cnteq < n_keep))
            x_v[v] = jnp.where(keep, xv, jnp.float32(_NEG))
            return cnteq + jnp.where(eq, 1, 0)

        pltpu.sync_copy(x_v, out_hbm.at[b, :, pl.ds(s0, _LN)])
        return carry

    lax.fori_loop(0, _SC_JPW, do_job, 0)


def _topk_mask_sc(logits):
    mesh = plsc.VectorSubcoreMesh(core_axis_name="c", subcore_axis_name="s")
    fn = functools.partial(
        pl.kernel,
        mesh=mesh,
        out_type=jax.ShapeDtypeStruct((_B_SC, _V, _S), jnp.float32),
        scratch_types=[pltpu.VMEM((_V, _LN), jnp.float32),
                       pltpu.VMEM((_NBINS, _LN), jnp.int32)],
        compiler_params=pltpu.CompilerParams(use_tc_tiling_on_sc=False,
                                             needs_layout_passes=False),
    )(_sc_body)
    return fn(logits)


@jax.jit
def _topk_mask(logits):
    sc_out = _topk_mask_sc(logits)
    tc_out = _topk_mask_tc(logits)
    return lax.dynamic_update_slice(tc_out, sc_out, (0, 0, 0))


def kernel(logits, k):
    del k
    return _topk_mask(logits)
